# baseline (speedup 1.0000x reference)
import functools

import jax
import jax.numpy as jnp
from jax import lax
from jax.experimental import pallas as pl
from jax.experimental.pallas import tpu as pltpu
from jax.experimental.pallas import tpu_sc as plsc

_EMB = 128
_DIMS = (81, 8, 12, 12, 10, 6, 2)
_VTOT = 131
_VPAD = 136
_NCOMB = 232
_NC = 2
_NS = 16
_NW = _NC * _NS
_CH = 224


def _splat(val):
    return jnp.full((16,), val, jnp.int32)


def _body(wcat_hbm, xt_hbm, out_hbm, tbl_v, ptbl_v, pcomb_v, idx0_v, idx1_v,
          cidx_v, out_v0, out_v1, isem, osem, n, bpw, nch):
    cid = lax.axis_index("c")
    sid = lax.axis_index("s")
    wid = sid * _NC + cid
    base = wid * bpw

    def stage_idx(start, dst):
        for i in range(7):
            pltpu.async_copy(
                xt_hbm.at[pl.ds(i * n + start, _CH)],
                dst.at[pl.ds(i * _CH, _CH)], isem,
            )

    def wait_idx(dst):
        for i in range(7):
            pltpu.make_async_copy(
                xt_hbm.at[pl.ds(0, _CH)], dst.at[pl.ds(i * _CH, _CH)], isem
            ).wait()

    stage_idx(base, idx0_v)
    pltpu.sync_copy(wcat_hbm, tbl_v)

    col0 = lax.iota(jnp.int32, 16)
    cols = [col0 + 16 * cc for cc in range(8)]

    def pack_store(dst, r, g, lo, hi):
        pw = plsc.bitcast(
            plsc.pack(lo, hi, format=plsc.PackFormat.INTERLEAVED), jnp.int32
        )
        dst[r, pl.ds(16 * g, 16)] = pw

    @plsc.parallel_loop(0, _VPAD, unroll=2)
    def pack_tbl(r):
        sr = _splat(0) + r
        for g in range(4):
            lo = plsc.load_gather(tbl_v, [sr, cols[g]])
            hi = plsc.load_gather(tbl_v, [sr, cols[g + 4]])
            pack_store(ptbl_v, r, g, lo, hi)

    def build(dst_off, src1_off, d1, src2_off, d2):
        def outer(a, _):
            sa = _splat(src1_off + a)

            @plsc.parallel_loop(0, d2, unroll=2)
            def inner(b):
                sb = _splat(src2_off + b)
                r = dst_off + a * d2 + b
                for g in range(4):
                    lo = plsc.load_gather(tbl_v, [sa, cols[g]]) + plsc.load_gather(
                        tbl_v, [sb, cols[g]]
                    )
                    hi = plsc.load_gather(
                        tbl_v, [sa, cols[g + 4]]
                    ) + plsc.load_gather(tbl_v, [sb, cols[g + 4]])
                    pack_store(pcomb_v, r, g, lo, hi)

            return 0

        lax.fori_loop(0, d1, outer, 0)

    build(0, 81, 8, 89, 12)
    build(96, 101, 12, 113, 10)
    build(216, 123, 6, 129, 2)

    def do_chunk(ci, idx_buf, nidx_buf, out_buf):
        wait_idx(idx_buf)

        @pl.when(ci + 1 < nch)
        def _():
            nstart = jnp.minimum(base + (ci + 1) * _CH, n - _CH)
            stage_idx(nstart, nidx_buf)

        @pl.when(ci >= 2)
        def _():
            pltpu.make_async_copy(
                out_hbm.at[pl.ds(0, _CH)], out_buf, osem
            ).wait()

        @plsc.parallel_loop(0, _CH // 16, unroll=2)
        def idx_body(g):
            g0 = g * 16
            xs = [idx_buf[pl.ds(i * _CH + g0, 16)] for i in range(7)]
            cidx_v[pl.ds(g0, 16)] = xs[0]
            cidx_v[pl.ds(_CH + g0, 16)] = xs[1] * 12 + xs[2]
            cidx_v[pl.ds(2 * _CH + g0, 16)] = xs[3] * 10 + xs[4] + 96
            cidx_v[pl.ds(3 * _CH + g0, 16)] = xs[5] * 2 + xs[6] + 216

        @plsc.parallel_loop(0, _CH, unroll=8)
        def row_body(j):
            rvec = _splat(0) + j
            sA = plsc.load_gather(cidx_v, [rvec])
            sB = plsc.load_gather(cidx_v, [rvec + _CH])
            sC = plsc.load_gather(cidx_v, [rvec + 2 * _CH])
            sD = plsc.load_gather(cidx_v, [rvec + 3 * _CH])
            for g in range(4):
                a = plsc.bitcast(
                    plsc.load_gather(ptbl_v, [sA, cols[g]]), jnp.bfloat16
                )
                b = plsc.bitcast(
                    plsc.load_gather(pcomb_v, [sB, cols[g]]), jnp.bfloat16
                )
                c = plsc.bitcast(
                    plsc.load_gather(pcomb_v, [sC, cols[g]]), jnp.bfloat16
                )
                d = plsc.bitcast(
                    plsc.load_gather(pcomb_v, [sD, cols[g]]), jnp.bfloat16
                )
                s = (a + b) + (c + d)
                lo, hi = plsc.unpack(s, format=plsc.PackFormat.INTERLEAVED)
                out_buf[j, pl.ds(16 * g, 16)] = lo
                out_buf[j, pl.ds(64 + 16 * g, 16)] = hi

        gstart = jnp.minimum(base + ci * _CH, n - _CH)
        pltpu.async_copy(out_buf, out_hbm.at[pl.ds(gstart, _CH)], osem)

    def chunk_pair(ci2, _):
        do_chunk(2 * ci2, idx0_v, idx1_v, out_v0)
        do_chunk(2 * ci2 + 1, idx1_v, idx0_v, out_v1)
        return 0

    lax.fori_loop(0, nch // 2, chunk_pair, 0)

    for _ in range(2):
        pltpu.make_async_copy(out_hbm.at[pl.ds(0, _CH)], out_v0, osem).wait()


def kernel(x, W0, W1, W2, W3, W4, W5, W6):
    n = x.shape[0]
    bpw = ((n + _NW * _CH - 1) // (_NW * _CH)) * _CH
    nch = bpw // _CH

    wcat = jnp.concatenate([W0, W1, W2, W3, W4, W5, W6], axis=0)
    wcat = jnp.pad(wcat, ((0, _VPAD - _VTOT), (0, 0)))
    xt = jnp.transpose(x).reshape(-1)

    mesh = plsc.VectorSubcoreMesh(core_axis_name="c", subcore_axis_name="s")
    fn = pl.kernel(
        functools.partial(_body, n=n, bpw=bpw, nch=nch),
        out_type=jax.ShapeDtypeStruct((n, _EMB), jnp.float32),
        mesh=mesh,
        compiler_params=pltpu.CompilerParams(needs_layout_passes=False),
        scratch_types=[
            pltpu.VMEM((_VPAD, _EMB), jnp.float32),
            pltpu.VMEM((_VPAD, _EMB // 2), jnp.int32),
            pltpu.VMEM((_NCOMB, _EMB // 2), jnp.int32),
            pltpu.VMEM((7 * _CH,), jnp.int32),
            pltpu.VMEM((7 * _CH,), jnp.int32),
            pltpu.VMEM((4 * _CH,), jnp.int32),
            pltpu.VMEM((_CH, _EMB), jnp.float32),
            pltpu.VMEM((_CH, _EMB), jnp.float32),
            pltpu.SemaphoreType.DMA,
            pltpu.SemaphoreType.DMA,
        ],
    )
    return fn(wcat, xt)


# speedup vs baseline: 1.0274x; 1.0274x over previous
import functools

import jax
import jax.numpy as jnp
from jax import lax
from jax.experimental import pallas as pl
from jax.experimental.pallas import tpu as pltpu
from jax.experimental.pallas import tpu_sc as plsc

_EMB = 128
_DIMS = (81, 8, 12, 12, 10, 6, 2)
_VTOT = 131
_VPAD = 136
_NCOMB = 232
_NC = 2
_NS = 16
_NW = _NC * _NS
_CH = 224


def _splat(val):
    return jnp.full((16,), val, jnp.int32)


def _body(wcat_hbm, xt_hbm, out_hbm, tbl_v, ptbl_v, pcomb_v, idx0_v, idx1_v,
          cidx_v, out_v0, out_v1, isem, osem, n, bpw, nch):
    cid = lax.axis_index("c")
    sid = lax.axis_index("s")
    wid = sid * _NC + cid
    base = wid * bpw

    def stage_idx(start, dst):
        for i in range(7):
            pltpu.async_copy(
                xt_hbm.at[pl.ds(i * n + start, _CH)],
                dst.at[pl.ds(i * _CH, _CH)], isem,
            )

    def wait_idx(dst):
        for i in range(7):
            pltpu.make_async_copy(
                xt_hbm.at[pl.ds(0, _CH)], dst.at[pl.ds(i * _CH, _CH)], isem
            ).wait()

    stage_idx(base, idx0_v)
    pltpu.sync_copy(wcat_hbm, tbl_v)

    col0 = lax.iota(jnp.int32, 16)
    cols = [col0 + 16 * cc for cc in range(8)]

    def pack_store(dst, r, g, lo, hi):
        pw = plsc.bitcast(
            plsc.pack(lo, hi, format=plsc.PackFormat.INTERLEAVED), jnp.int32
        )
        dst[r, pl.ds(16 * g, 16)] = pw

    @plsc.parallel_loop(0, _VPAD, unroll=2)
    def pack_tbl(r):
        sr = _splat(0) + r
        for g in range(4):
            lo = plsc.load_gather(tbl_v, [sr, cols[g]])
            hi = plsc.load_gather(tbl_v, [sr, cols[g + 4]])
            pack_store(ptbl_v, r, g, lo, hi)

    def build(dst_off, src1_off, d1, src2_off, d2):
        def outer(a, _):
            sa = _splat(src1_off + a)

            @plsc.parallel_loop(0, d2, unroll=2)
            def inner(b):
                sb = _splat(src2_off + b)
                r = dst_off + a * d2 + b
                for g in range(4):
                    lo = plsc.load_gather(tbl_v, [sa, cols[g]]) + plsc.load_gather(
                        tbl_v, [sb, cols[g]]
                    )
                    hi = plsc.load_gather(
                        tbl_v, [sa, cols[g + 4]]
                    ) + plsc.load_gather(tbl_v, [sb, cols[g + 4]])
                    pack_store(pcomb_v, r, g, lo, hi)

            return 0

        lax.fori_loop(0, d1, outer, 0)

    build(0, 81, 8, 89, 12)
    build(96, 101, 12, 113, 10)
    build(216, 123, 6, 129, 2)

    def do_chunk(ci, idx_buf, nidx_buf, out_buf):
        wait_idx(idx_buf)

        @pl.when(ci + 1 < nch)
        def _():
            nstart = jnp.minimum(base + (ci + 1) * _CH, n - _CH)
            stage_idx(nstart, nidx_buf)

        @pl.when(ci >= 2)
        def _():
            pltpu.make_async_copy(
                out_hbm.at[pl.ds(0, _CH)], out_buf, osem
            ).wait()

        @plsc.parallel_loop(0, _CH // 16, unroll=2)
        def idx_body(g):
            g0 = g * 16
            xs = [idx_buf[pl.ds(i * _CH + g0, 16)] for i in range(7)]
            cidx_v[pl.ds(g0, 16)] = xs[0]
            cidx_v[pl.ds(_CH + g0, 16)] = xs[1] * 12 + xs[2]
            cidx_v[pl.ds(2 * _CH + g0, 16)] = xs[3] * 10 + xs[4] + 96
            cidx_v[pl.ds(3 * _CH + g0, 16)] = xs[5] * 2 + xs[6] + 216

        @plsc.parallel_loop(0, _CH, unroll=6)
        def row_body(j):
            rvec = _splat(0) + j
            sA = plsc.load_gather(cidx_v, [rvec])
            sB = plsc.load_gather(cidx_v, [rvec + _CH])
            sC = plsc.load_gather(cidx_v, [rvec + 2 * _CH])
            sD = plsc.load_gather(cidx_v, [rvec + 3 * _CH])
            for g in range(4):
                a = plsc.bitcast(
                    plsc.load_gather(ptbl_v, [sA, cols[g]]), jnp.bfloat16
                )
                b = plsc.bitcast(
                    plsc.load_gather(pcomb_v, [sB, cols[g]]), jnp.bfloat16
                )
                c = plsc.bitcast(
                    plsc.load_gather(pcomb_v, [sC, cols[g]]), jnp.bfloat16
                )
                d = plsc.bitcast(
                    plsc.load_gather(pcomb_v, [sD, cols[g]]), jnp.bfloat16
                )
                s = (a + b) + (c + d)
                lo, hi = plsc.unpack(s, format=plsc.PackFormat.INTERLEAVED)
                out_buf[j, pl.ds(16 * g, 16)] = lo
                out_buf[j, pl.ds(64 + 16 * g, 16)] = hi

        gstart = jnp.minimum(base + ci * _CH, n - _CH)
        pltpu.async_copy(out_buf, out_hbm.at[pl.ds(gstart, _CH)], osem)

    def chunk_pair(ci2, _):
        do_chunk(2 * ci2, idx0_v, idx1_v, out_v0)
        do_chunk(2 * ci2 + 1, idx1_v, idx0_v, out_v1)
        return 0

    lax.fori_loop(0, nch // 2, chunk_pair, 0)

    for _ in range(2):
        pltpu.make_async_copy(out_hbm.at[pl.ds(0, _CH)], out_v0, osem).wait()


def kernel(x, W0, W1, W2, W3, W4, W5, W6):
    n = x.shape[0]
    bpw = ((n + _NW * _CH - 1) // (_NW * _CH)) * _CH
    nch = bpw // _CH

    wcat = jnp.concatenate([W0, W1, W2, W3, W4, W5, W6], axis=0)
    wcat = jnp.pad(wcat, ((0, _VPAD - _VTOT), (0, 0)))
    xt = jnp.transpose(x).reshape(-1)

    mesh = plsc.VectorSubcoreMesh(core_axis_name="c", subcore_axis_name="s")
    fn = pl.kernel(
        functools.partial(_body, n=n, bpw=bpw, nch=nch),
        out_type=jax.ShapeDtypeStruct((n, _EMB), jnp.float32),
        mesh=mesh,
        compiler_params=pltpu.CompilerParams(needs_layout_passes=False),
        scratch_types=[
            pltpu.VMEM((_VPAD, _EMB), jnp.float32),
            pltpu.VMEM((_VPAD, _EMB // 2), jnp.int32),
            pltpu.VMEM((_NCOMB, _EMB // 2), jnp.int32),
            pltpu.VMEM((7 * _CH,), jnp.int32),
            pltpu.VMEM((7 * _CH,), jnp.int32),
            pltpu.VMEM((4 * _CH,), jnp.int32),
            pltpu.VMEM((_CH, _EMB), jnp.float32),
            pltpu.VMEM((_CH, _EMB), jnp.float32),
            pltpu.SemaphoreType.DMA,
            pltpu.SemaphoreType.DMA,
        ],
    )
    return fn(wcat, xt)


# speedup vs baseline: 1.0290x; 1.0015x over previous
import functools

import jax
import jax.numpy as jnp
from jax import lax
from jax.experimental import pallas as pl
from jax.experimental.pallas import tpu as pltpu
from jax.experimental.pallas import tpu_sc as plsc

_EMB = 128
_DIMS = (81, 8, 12, 12, 10, 6, 2)
_VTOT = 131
_VPAD = 136
_NCOMB = 232
_NC = 2
_NS = 16
_NW = _NC * _NS
_CH = 224


def _splat(val):
    return jnp.full((16,), val, jnp.int32)


def _body(wcat_hbm, xt_hbm, out_hbm, tbl_v, ptbl_v, pcomb_v, idx0_v, idx1_v,
          cidx_v, out_v0, out_v1, isem, osem, n, bpw, nch):
    cid = lax.axis_index("c")
    sid = lax.axis_index("s")
    wid = sid * _NC + cid
    base = wid * bpw

    def stage_idx(start, dst):
        for i in range(7):
            pltpu.async_copy(
                xt_hbm.at[pl.ds(i * n + start, _CH)],
                dst.at[pl.ds(i * _CH, _CH)], isem,
            )

    def wait_idx(dst):
        for i in range(7):
            pltpu.make_async_copy(
                xt_hbm.at[pl.ds(0, _CH)], dst.at[pl.ds(i * _CH, _CH)], isem
            ).wait()

    stage_idx(base, idx0_v)
    pltpu.sync_copy(wcat_hbm, tbl_v)

    col0 = lax.iota(jnp.int32, 16)
    cols = [col0 + 16 * cc for cc in range(8)]

    def pack_store(dst, r, g, lo, hi):
        pw = plsc.bitcast(
            plsc.pack(lo, hi, format=plsc.PackFormat.INTERLEAVED), jnp.int32
        )
        dst[r, pl.ds(16 * g, 16)] = pw

    @plsc.parallel_loop(0, _VPAD, unroll=1)
    def pack_tbl(r):
        sr = _splat(0) + r
        for g in range(4):
            lo = plsc.load_gather(tbl_v, [sr, cols[g]])
            hi = plsc.load_gather(tbl_v, [sr, cols[g + 4]])
            pack_store(ptbl_v, r, g, lo, hi)

    def build(dst_off, src1_off, d1, src2_off, d2):
        def outer(a, _):
            sa = _splat(src1_off + a)

            @plsc.parallel_loop(0, d2, unroll=1)
            def inner(b):
                sb = _splat(src2_off + b)
                r = dst_off + a * d2 + b
                for g in range(4):
                    lo = plsc.load_gather(tbl_v, [sa, cols[g]]) + plsc.load_gather(
                        tbl_v, [sb, cols[g]]
                    )
                    hi = plsc.load_gather(
                        tbl_v, [sa, cols[g + 4]]
                    ) + plsc.load_gather(tbl_v, [sb, cols[g + 4]])
                    pack_store(pcomb_v, r, g, lo, hi)

            return 0

        lax.fori_loop(0, d1, outer, 0)

    build(0, 81, 8, 89, 12)
    build(96, 101, 12, 113, 10)
    build(216, 123, 6, 129, 2)

    def do_chunk(ci, idx_buf, nidx_buf, out_buf):
        wait_idx(idx_buf)

        @pl.when(ci + 1 < nch)
        def _():
            nstart = jnp.minimum(base + (ci + 1) * _CH, n - _CH)
            stage_idx(nstart, nidx_buf)

        @pl.when(ci >= 2)
        def _():
            pltpu.make_async_copy(
                out_hbm.at[pl.ds(0, _CH)], out_buf, osem
            ).wait()

        @plsc.parallel_loop(0, _CH // 16, unroll=2)
        def idx_body(g):
            g0 = g * 16
            xs = [idx_buf[pl.ds(i * _CH + g0, 16)] for i in range(7)]
            cidx_v[pl.ds(g0, 16)] = xs[0]
            cidx_v[pl.ds(_CH + g0, 16)] = xs[1] * 12 + xs[2]
            cidx_v[pl.ds(2 * _CH + g0, 16)] = xs[3] * 10 + xs[4] + 96
            cidx_v[pl.ds(3 * _CH + g0, 16)] = xs[5] * 2 + xs[6] + 216

        @plsc.parallel_loop(0, _CH, unroll=6)
        def row_body(j):
            rvec = _splat(0) + j
            sA = plsc.load_gather(cidx_v, [rvec])
            sB = plsc.load_gather(cidx_v, [rvec + _CH])
            sC = plsc.load_gather(cidx_v, [rvec + 2 * _CH])
            sD = plsc.load_gather(cidx_v, [rvec + 3 * _CH])
            for g in range(4):
                a = plsc.bitcast(
                    plsc.load_gather(ptbl_v, [sA, cols[g]]), jnp.bfloat16
                )
                b = plsc.bitcast(
                    plsc.load_gather(pcomb_v, [sB, cols[g]]), jnp.bfloat16
                )
                c = plsc.bitcast(
                    plsc.load_gather(pcomb_v, [sC, cols[g]]), jnp.bfloat16
                )
                d = plsc.bitcast(
                    plsc.load_gather(pcomb_v, [sD, cols[g]]), jnp.bfloat16
                )
                s = (a + b) + (c + d)
                lo, hi = plsc.unpack(s, format=plsc.PackFormat.INTERLEAVED)
                out_buf[j, pl.ds(16 * g, 16)] = lo
                out_buf[j, pl.ds(64 + 16 * g, 16)] = hi

        gstart = jnp.minimum(base + ci * _CH, n - _CH)
        pltpu.async_copy(out_buf, out_hbm.at[pl.ds(gstart, _CH)], osem)

    def chunk_pair(ci2, _):
        do_chunk(2 * ci2, idx0_v, idx1_v, out_v0)
        do_chunk(2 * ci2 + 1, idx1_v, idx0_v, out_v1)
        return 0

    lax.fori_loop(0, nch // 2, chunk_pair, 0)

    for _ in range(2):
        pltpu.make_async_copy(out_hbm.at[pl.ds(0, _CH)], out_v0, osem).wait()


def kernel(x, W0, W1, W2, W3, W4, W5, W6):
    n = x.shape[0]
    bpw = ((n + _NW * _CH - 1) // (_NW * _CH)) * _CH
    nch = bpw // _CH

    wcat = jnp.concatenate([W0, W1, W2, W3, W4, W5, W6], axis=0)
    wcat = jnp.pad(wcat, ((0, _VPAD - _VTOT), (0, 0)))
    xt = jnp.transpose(x).reshape(-1)

    mesh = plsc.VectorSubcoreMesh(core_axis_name="c", subcore_axis_name="s")
    fn = pl.kernel(
        functools.partial(_body, n=n, bpw=bpw, nch=nch),
        out_type=jax.ShapeDtypeStruct((n, _EMB), jnp.float32),
        mesh=mesh,
        compiler_params=pltpu.CompilerParams(needs_layout_passes=False),
        scratch_types=[
            pltpu.VMEM((_VPAD, _EMB), jnp.float32),
            pltpu.VMEM((_VPAD, _EMB // 2), jnp.int32),
            pltpu.VMEM((_NCOMB, _EMB // 2), jnp.int32),
            pltpu.VMEM((7 * _CH,), jnp.int32),
            pltpu.VMEM((7 * _CH,), jnp.int32),
            pltpu.VMEM((4 * _CH,), jnp.int32),
            pltpu.VMEM((_CH, _EMB), jnp.float32),
            pltpu.VMEM((_CH, _EMB), jnp.float32),
            pltpu.SemaphoreType.DMA,
            pltpu.SemaphoreType.DMA,
        ],
    )
    return fn(wcat, xt)


# speedup vs baseline: 1.0705x; 1.0404x over previous
import functools

import jax
import jax.numpy as jnp
from jax import lax
from jax.experimental import pallas as pl
from jax.experimental.pallas import tpu as pltpu
from jax.experimental.pallas import tpu_sc as plsc

_EMB = 128
_DIMS = (81, 8, 12, 12, 10, 6, 2)
_VTOT = 131
_VPAD = 136
_NCOMB = 232
_NC = 2
_NS = 16
_NW = _NC * _NS
_CH = 224


def _splat(val):
    return jnp.full((16,), val, jnp.int32)


def _body(wcat_hbm, xt_hbm, out_hbm, tbl_v, ptbl_v, pcomb_v, idx0_v, idx1_v,
          cidx_v, out_v0, out_v1, isem, osem, n, bpw, nch):
    cid = lax.axis_index("c")
    sid = lax.axis_index("s")
    wid = sid * _NC + cid
    base = wid * bpw

    def stage_idx(start, dst):
        for i in range(7):
            pltpu.async_copy(
                xt_hbm.at[pl.ds(i * n + start, _CH)],
                dst.at[pl.ds(i * _CH, _CH)], isem,
            )

    def wait_idx(dst):
        for i in range(7):
            pltpu.make_async_copy(
                xt_hbm.at[pl.ds(0, _CH)], dst.at[pl.ds(i * _CH, _CH)], isem
            ).wait()

    stage_idx(base, idx0_v)
    pltpu.sync_copy(wcat_hbm, tbl_v)

    col0 = lax.iota(jnp.int32, 16)
    cols = [col0 + 16 * cc for cc in range(8)]

    def pack_store(dst, r, g, lo, hi):
        pw = plsc.bitcast(
            plsc.pack(lo, hi, format=plsc.PackFormat.INTERLEAVED), jnp.int32
        )
        dst[r, pl.ds(16 * g, 16)] = pw

    @plsc.parallel_loop(0, _VPAD, unroll=1)
    def pack_tbl(r):
        sr = _splat(0) + r
        for g in range(4):
            lo = plsc.load_gather(tbl_v, [sr, cols[g]])
            hi = plsc.load_gather(tbl_v, [sr, cols[g + 4]])
            pack_store(ptbl_v, r, g, lo, hi)

    def build(dst_off, src1_off, d1, src2_off, d2):
        def outer(a, _):
            sa = _splat(src1_off + a)

            @plsc.parallel_loop(0, d2, unroll=1)
            def inner(b):
                sb = _splat(src2_off + b)
                r = dst_off + a * d2 + b
                for g in range(4):
                    lo = plsc.load_gather(tbl_v, [sa, cols[g]]) + plsc.load_gather(
                        tbl_v, [sb, cols[g]]
                    )
                    hi = plsc.load_gather(
                        tbl_v, [sa, cols[g + 4]]
                    ) + plsc.load_gather(tbl_v, [sb, cols[g + 4]])
                    pack_store(pcomb_v, r, g, lo, hi)

            return 0

        lax.fori_loop(0, d1, outer, 0)

    build(0, 81, 8, 89, 12)
    build(96, 101, 12, 113, 10)
    build(216, 123, 6, 129, 2)

    def do_chunk(ci, idx_buf, nidx_buf, out_buf):
        wait_idx(idx_buf)

        @pl.when(ci + 1 < nch)
        def _():
            nstart = jnp.minimum(base + (ci + 1) * _CH, n - _CH)
            stage_idx(nstart, nidx_buf)

        @pl.when(ci >= 2)
        def _():
            pltpu.make_async_copy(
                out_hbm.at[pl.ds(0, _CH)], out_buf, osem
            ).wait()

        @plsc.parallel_loop(0, _CH // 16, unroll=2)
        def idx_body(g):
            g0 = g * 16
            xs = [idx_buf[pl.ds(i * _CH + g0, 16)] for i in range(7)]
            ib = xs[1] * 12 + xs[2]
            idd = xs[5] * 2 + xs[6] + 216
            cidx_v[pl.ds(g0, 16)] = xs[0] | (ib << 16)
            cidx_v[pl.ds(_CH + g0, 16)] = (xs[3] * 10 + xs[4] + 96) | (idd << 16)

        @plsc.parallel_loop(0, _CH, unroll=6)
        def row_body(j):
            rvec = _splat(0) + j
            sAB = plsc.load_gather(cidx_v, [rvec])
            sCD = plsc.load_gather(cidx_v, [rvec + _CH])
            sA = sAB & 0xFFFF
            sB = lax.shift_right_logical(sAB, 16)
            sC = sCD & 0xFFFF
            sD = lax.shift_right_logical(sCD, 16)
            for g in range(4):
                a = plsc.bitcast(
                    plsc.load_gather(ptbl_v, [sA, cols[g]]), jnp.bfloat16
                )
                b = plsc.bitcast(
                    plsc.load_gather(pcomb_v, [sB, cols[g]]), jnp.bfloat16
                )
                c = plsc.bitcast(
                    plsc.load_gather(pcomb_v, [sC, cols[g]]), jnp.bfloat16
                )
                d = plsc.bitcast(
                    plsc.load_gather(pcomb_v, [sD, cols[g]]), jnp.bfloat16
                )
                s = (a + b) + (c + d)
                lo, hi = plsc.unpack(s, format=plsc.PackFormat.INTERLEAVED)
                out_buf[j, pl.ds(16 * g, 16)] = lo
                out_buf[j, pl.ds(64 + 16 * g, 16)] = hi

        gstart = jnp.minimum(base + ci * _CH, n - _CH)
        pltpu.async_copy(out_buf, out_hbm.at[pl.ds(gstart, _CH)], osem)

    def chunk_pair(ci2, _):
        do_chunk(2 * ci2, idx0_v, idx1_v, out_v0)
        do_chunk(2 * ci2 + 1, idx1_v, idx0_v, out_v1)
        return 0

    lax.fori_loop(0, nch // 2, chunk_pair, 0)

    for _ in range(2):
        pltpu.make_async_copy(out_hbm.at[pl.ds(0, _CH)], out_v0, osem).wait()


def kernel(x, W0, W1, W2, W3, W4, W5, W6):
    n = x.shape[0]
    bpw = ((n + _NW * _CH - 1) // (_NW * _CH)) * _CH
    nch = bpw // _CH

    wcat = jnp.concatenate([W0, W1, W2, W3, W4, W5, W6], axis=0)
    wcat = jnp.pad(wcat, ((0, _VPAD - _VTOT), (0, 0)))
    xt = jnp.transpose(x).reshape(-1)

    mesh = plsc.VectorSubcoreMesh(core_axis_name="c", subcore_axis_name="s")
    fn = pl.kernel(
        functools.partial(_body, n=n, bpw=bpw, nch=nch),
        out_type=jax.ShapeDtypeStruct((n, _EMB), jnp.float32),
        mesh=mesh,
        compiler_params=pltpu.CompilerParams(needs_layout_passes=False),
        scratch_types=[
            pltpu.VMEM((_VPAD, _EMB), jnp.float32),
            pltpu.VMEM((_VPAD, _EMB // 2), jnp.int32),
            pltpu.VMEM((_NCOMB, _EMB // 2), jnp.int32),
            pltpu.VMEM((7 * _CH,), jnp.int32),
            pltpu.VMEM((7 * _CH,), jnp.int32),
            pltpu.VMEM((4 * _CH,), jnp.int32),
            pltpu.VMEM((_CH, _EMB), jnp.float32),
            pltpu.VMEM((_CH, _EMB), jnp.float32),
            pltpu.SemaphoreType.DMA,
            pltpu.SemaphoreType.DMA,
        ],
    )
    return fn(wcat, xt)
